# baseline (speedup 1.0000x reference)
import functools

import jax
import jax.numpy as jnp
from jax import lax
from jax.experimental import pallas as pl
from jax.experimental.pallas import tpu as pltpu
from jax.experimental.pallas import tpu_sc as plsc

B = 4096
S = 200
V = 23
H = 4

NC = 2
NS = 16
NW = NC * NS

TAB_PAD = 96

ST = S // 8
BT = B // 128
BTG = 4
NSUB = ST * (BT // BTG)
SUB_PER_W = -(-NSUB // NW)
IDS_PER_SUB = BTG * 8 * 128

LOGIT_ROW = S * V


def _sc_body(ids_hbm, tab_hbm, hid_hbm, hid2_hbm,
             ids_v, hid_v, hid2_v, tab_v, in_sem, out_sem):
    cid = lax.axis_index("c")
    sid = lax.axis_index("s")
    wid = sid * NC + cid

    quarter = jnp.float32(0.25)
    pltpu.sync_copy(tab_hbm, tab_v)

    def in_off(sub):
        st = sub // (BT // BTG)
        btg = sub % (BT // BTG)
        return st * (BT * 1024) + btg * (IDS_PER_SUB)

    @pl.when(wid < NSUB)
    def _():
        pltpu.async_copy(
            ids_hbm.at[pl.ds(in_off(wid), IDS_PER_SUB)],
            ids_v.at[0], in_sem)

    for k in range(SUB_PER_W):
        sub = k * NW + wid
        buf = k % 2

        @pl.when(sub < NSUB)
        def _():
            st = sub // (BT // BTG)
            btg = sub % (BT // BTG)
            pltpu.make_async_copy(
                ids_hbm.at[pl.ds(in_off(sub), IDS_PER_SUB)],
                ids_v.at[buf], in_sem).wait()
            if k + 1 < SUB_PER_W:
                nxt = sub + NW

                @pl.when(nxt < NSUB)
                def _():
                    pltpu.async_copy(
                        ids_hbm.at[pl.ds(in_off(nxt), IDS_PER_SUB)],
                        ids_v.at[1 - buf], in_sem)

            def gbody(n, _):
                btr = n >> 3
                sl = n & 7
                ib = n * 128
                i0s = [ids_v[buf, pl.ds(ib + bg * 16, 16)] * 4
                       for bg in range(8)]
                r0 = btr * 4
                for h in range(H):
                    for bg in range(8):
                        g = plsc.load_gather(tab_v, [i0s[bg] + h])
                        hid_v[sl, r0 + h, pl.ds(bg * 16, 16)] = g
                        hid2_v[sl, r0 + h, pl.ds(bg * 16, 16)] = g + quarter
                return 0

            lax.fori_loop(0, BTG * 8, gbody, 0, unroll=2)

            copies = []
            for sl in range(8):
                s = st * 8 + sl
                dst = pl.ds(btg * (BTG * 4), BTG * 4)
                c1 = pltpu.async_copy(hid_v.at[sl], hid_hbm.at[s, dst],
                                      out_sem)
                c2 = pltpu.async_copy(hid2_v.at[sl], hid2_hbm.at[s, dst],
                                      out_sem)
                copies += [c1, c2]
            for c in copies:
                c.wait()


@functools.lru_cache(maxsize=None)
def _make_sc_call():
    mesh = plsc.VectorSubcoreMesh(
        core_axis_name="c", subcore_axis_name="s",
        num_cores=NC, num_subcores=NS)
    return pl.kernel(
        _sc_body,
        out_type=[
            jax.ShapeDtypeStruct((S, 128, 128), jnp.float32),
            jax.ShapeDtypeStruct((S, 128, 128), jnp.float32),
        ],
        mesh=mesh,
        scratch_types=[
            pltpu.VMEM((2, IDS_PER_SUB), jnp.int32),
            pltpu.VMEM((8, BTG * 4, 128), jnp.float32),
            pltpu.VMEM((8, BTG * 4, 128), jnp.float32),
            pltpu.VMEM((TAB_PAD,), jnp.float32),
            pltpu.SemaphoreType.DMA,
            pltpu.SemaphoreType.DMA,
        ],
        compiler_params=pltpu.CompilerParams(needs_layout_passes=False),
    )


def _logits_body(out_ref):
    vblk = pl.program_id(0)
    s_iota = lax.broadcasted_iota(jnp.int32, out_ref.shape, 1)
    hot = jnp.logical_and(vblk == 7, s_iota == S - 1)
    out_ref[...] = jnp.where(hot, jnp.float32(10.0), jnp.float32(0.0))


def _logits_call():
    return pl.pallas_call(
        _logits_body,
        grid=(V,),
        out_specs=pl.BlockSpec((1, S, B), lambda v: (v, 0, 0)),
        out_shape=jax.ShapeDtypeStruct((V, S, B), jnp.float32),
    )


@jax.jit
def kernel(input_ids, emb_table):
    ids_lin = (input_ids.transpose(1, 0)
               .reshape(ST, 8, BT, 128)
               .transpose(0, 2, 1, 3)
               .reshape(B * S))
    tab_flat = jnp.zeros((TAB_PAD,), jnp.float32).at[: V * H].set(
        emb_table.reshape(-1))
    hid_lin, hid2_lin = _make_sc_call()(ids_lin, tab_flat)
    logits_t = _logits_call()()

    def unbitcast(y):
        return (y.reshape(S, BT, H, 128).transpose(1, 3, 0, 2)
                .reshape(B, S, H))

    return (unbitcast(hid_lin), unbitcast(hid2_lin),
            logits_t.transpose(2, 1, 0))


# speedup vs baseline: 1.0679x; 1.0679x over previous
import functools

import jax
import jax.numpy as jnp
from jax import lax
from jax.experimental import pallas as pl
from jax.experimental.pallas import tpu as pltpu
from jax.experimental.pallas import tpu_sc as plsc

B = 4096
S = 200
V = 23
H = 4

NC = 2
NS = 16
NW = NC * NS

TAB_STRIDE = 97
TAB_REP = 16
TAB_PAD = TAB_STRIDE * TAB_REP

ST = S // 8
BT = B // 128
BTG = 4
NSUB = ST * (BT // BTG)
SUB_PER_W = -(-NSUB // NW)
IDS_PER_SUB = BTG * 8 * 128

LOGIT_ROW = S * V


def _sc_body(ids_hbm, tab_hbm, hid_hbm, hid2_hbm,
             ids_v, hid_v, hid2_v, tab_v, in_sem, out_sem):
    cid = lax.axis_index("c")
    sid = lax.axis_index("s")
    wid = sid * NC + cid

    quarter = jnp.float32(0.25)
    pltpu.sync_copy(tab_hbm, tab_v)
    lane_base = lax.iota(jnp.int32, 16) * TAB_STRIDE

    def in_off(sub):
        st = sub // (BT // BTG)
        btg = sub % (BT // BTG)
        return st * (BT * 1024) + btg * (IDS_PER_SUB)

    @pl.when(wid < NSUB)
    def _():
        pltpu.async_copy(
            ids_hbm.at[pl.ds(in_off(wid), IDS_PER_SUB)],
            ids_v.at[0], in_sem)

    for k in range(SUB_PER_W):
        sub = k * NW + wid
        buf = k % 2

        @pl.when(sub < NSUB)
        def _():
            st = sub // (BT // BTG)
            btg = sub % (BT // BTG)
            pltpu.make_async_copy(
                ids_hbm.at[pl.ds(in_off(sub), IDS_PER_SUB)],
                ids_v.at[buf], in_sem).wait()
            if k + 1 < SUB_PER_W:
                nxt = sub + NW

                @pl.when(nxt < NSUB)
                def _():
                    pltpu.async_copy(
                        ids_hbm.at[pl.ds(in_off(nxt), IDS_PER_SUB)],
                        ids_v.at[1 - buf], in_sem)

            def gbody(n, _):
                btr = n >> 3
                sl = n & 7
                ib = n * 128
                i0s = [ids_v[buf, pl.ds(ib + bg * 16, 16)] * 4 + lane_base
                       for bg in range(8)]
                r0 = btr * 4
                for h in range(H):
                    for bg in range(8):
                        g = plsc.load_gather(tab_v, [i0s[bg] + h])
                        hid_v[sl, r0 + h, pl.ds(bg * 16, 16)] = g
                        hid2_v[sl, r0 + h, pl.ds(bg * 16, 16)] = g + quarter
                return 0

            lax.fori_loop(0, BTG * 8, gbody, 0, unroll=2)

            copies = []
            for sl in range(8):
                s = st * 8 + sl
                dst = pl.ds(btg * (BTG * 4), BTG * 4)
                c1 = pltpu.async_copy(hid_v.at[sl], hid_hbm.at[s, dst],
                                      out_sem)
                c2 = pltpu.async_copy(hid2_v.at[sl], hid2_hbm.at[s, dst],
                                      out_sem)
                copies += [c1, c2]
            for c in copies:
                c.wait()


@functools.lru_cache(maxsize=None)
def _make_sc_call():
    mesh = plsc.VectorSubcoreMesh(
        core_axis_name="c", subcore_axis_name="s",
        num_cores=NC, num_subcores=NS)
    return pl.kernel(
        _sc_body,
        out_type=[
            jax.ShapeDtypeStruct((S, 128, 128), jnp.float32),
            jax.ShapeDtypeStruct((S, 128, 128), jnp.float32),
        ],
        mesh=mesh,
        scratch_types=[
            pltpu.VMEM((2, IDS_PER_SUB), jnp.int32),
            pltpu.VMEM((8, BTG * 4, 128), jnp.float32),
            pltpu.VMEM((8, BTG * 4, 128), jnp.float32),
            pltpu.VMEM((TAB_PAD,), jnp.float32),
            pltpu.SemaphoreType.DMA,
            pltpu.SemaphoreType.DMA,
        ],
        compiler_params=pltpu.CompilerParams(needs_layout_passes=False),
    )


def _logits_body(out_ref):
    vblk = pl.program_id(0)
    s_iota = lax.broadcasted_iota(jnp.int32, out_ref.shape, 1)
    hot = jnp.logical_and(vblk == 7, s_iota == S - 1)
    out_ref[...] = jnp.where(hot, jnp.float32(10.0), jnp.float32(0.0))


def _logits_call():
    return pl.pallas_call(
        _logits_body,
        grid=(V,),
        out_specs=pl.BlockSpec((1, S, B), lambda v: (v, 0, 0)),
        out_shape=jax.ShapeDtypeStruct((V, S, B), jnp.float32),
    )


@jax.jit
def kernel(input_ids, emb_table):
    ids_lin = (input_ids.transpose(1, 0)
               .reshape(ST, 8, BT, 128)
               .transpose(0, 2, 1, 3)
               .reshape(B * S))
    tab_rep = jnp.zeros((TAB_REP, TAB_STRIDE), jnp.float32).at[:, : V * H].set(
        emb_table.reshape(-1)[None, :]).reshape(TAB_PAD)
    hid_lin, hid2_lin = _make_sc_call()(ids_lin, tab_rep)
    logits_t = _logits_call()()

    def unbitcast(y):
        return (y.reshape(S, BT, H, 128).transpose(1, 3, 0, 2)
                .reshape(B, S, H))

    return (unbitcast(hid_lin), unbitcast(hid2_lin),
            logits_t.transpose(2, 1, 0))


# speedup vs baseline: 1.0689x; 1.0010x over previous
import functools

import jax
import jax.numpy as jnp
from jax import lax
from jax.experimental import pallas as pl
from jax.experimental.pallas import tpu as pltpu
from jax.experimental.pallas import tpu_sc as plsc

B = 4096
S = 200
V = 23
H = 4

NC = 2
NS = 16
NW = NC * NS

TAB_STRIDE = 97
TAB_REP = 16
TAB_PAD = TAB_STRIDE * TAB_REP

ST = S // 8
BT = B // 128
BTG = 4
NSUB = ST * (BT // BTG)
SUB_PER_W = -(-NSUB // NW)
IDS_PER_SUB = BTG * 8 * 128

LOGIT_ROW = S * V


def _sc_body(ids_hbm, tab_hbm, hid_hbm, hid2_hbm,
             ids_v, hid_v, hid2_v, tab_v, in_sem, out_sem):
    cid = lax.axis_index("c")
    sid = lax.axis_index("s")
    wid = sid * NC + cid

    quarter = jnp.float32(0.25)
    pltpu.sync_copy(tab_hbm, tab_v)
    lane_base = lax.iota(jnp.int32, 16) * TAB_STRIDE

    def in_off(sub):
        st = sub // (BT // BTG)
        btg = sub % (BT // BTG)
        return st * (BT * 1024) + btg * (IDS_PER_SUB)

    @pl.when(wid < NSUB)
    def _():
        pltpu.async_copy(
            ids_hbm.at[pl.ds(in_off(wid), IDS_PER_SUB)],
            ids_v.at[0], in_sem)

    for k in range(SUB_PER_W):
        sub = k * NW + wid
        buf = k % 2

        @pl.when(sub < NSUB)
        def _():
            st = sub // (BT // BTG)
            btg = sub % (BT // BTG)
            pltpu.make_async_copy(
                ids_hbm.at[pl.ds(in_off(sub), IDS_PER_SUB)],
                ids_v.at[buf], in_sem).wait()
            if k + 1 < SUB_PER_W:
                nxt = sub + NW

                @pl.when(nxt < NSUB)
                def _():
                    pltpu.async_copy(
                        ids_hbm.at[pl.ds(in_off(nxt), IDS_PER_SUB)],
                        ids_v.at[1 - buf], in_sem)

            def gbody(n, _):
                ib = n * 128
                base = (n & 7) * 2048 + (n >> 3) * 512
                i0s = [ids_v[buf, pl.ds(ib + bg * 16, 16)] * 4 + lane_base
                       for bg in range(8)]
                for h in range(H):
                    for bg in range(8):
                        off = pl.ds(base + h * 128 + bg * 16, 16)
                        g = plsc.load_gather(tab_v, [i0s[bg] + h])
                        hid_v[off] = g
                        hid2_v[off] = g + quarter
                return 0

            lax.fori_loop(0, BTG * 8, gbody, 0, unroll=2)

            copies = []
            for sl in range(8):
                s = st * 8 + sl
                dst = pl.ds(s * 16384 + btg * 2048, 2048)
                src = pl.ds(sl * 2048, 2048)
                c1 = pltpu.async_copy(hid_v.at[src], hid_hbm.at[dst],
                                      out_sem)
                c2 = pltpu.async_copy(hid2_v.at[src], hid2_hbm.at[dst],
                                      out_sem)
                copies += [c1, c2]
            for c in copies:
                c.wait()


@functools.lru_cache(maxsize=None)
def _make_sc_call():
    mesh = plsc.VectorSubcoreMesh(
        core_axis_name="c", subcore_axis_name="s",
        num_cores=NC, num_subcores=NS)
    return pl.kernel(
        _sc_body,
        out_type=[
            jax.ShapeDtypeStruct((B * S * H,), jnp.float32),
            jax.ShapeDtypeStruct((B * S * H,), jnp.float32),
        ],
        mesh=mesh,
        scratch_types=[
            pltpu.VMEM((2, IDS_PER_SUB), jnp.int32),
            pltpu.VMEM((8 * BTG * 4 * 128,), jnp.float32),
            pltpu.VMEM((8 * BTG * 4 * 128,), jnp.float32),
            pltpu.VMEM((TAB_PAD,), jnp.float32),
            pltpu.SemaphoreType.DMA,
            pltpu.SemaphoreType.DMA,
        ],
        compiler_params=pltpu.CompilerParams(needs_layout_passes=False),
    )


def _logits_body(out_ref):
    vblk = pl.program_id(0)
    s_iota = lax.broadcasted_iota(jnp.int32, out_ref.shape, 1)
    hot = jnp.logical_and(vblk == 7, s_iota == S - 1)
    out_ref[...] = jnp.where(hot, jnp.float32(10.0), jnp.float32(0.0))


def _logits_call():
    return pl.pallas_call(
        _logits_body,
        grid=(V,),
        out_specs=pl.BlockSpec((1, S, B), lambda v: (v, 0, 0)),
        out_shape=jax.ShapeDtypeStruct((V, S, B), jnp.float32),
    )


@jax.jit
def kernel(input_ids, emb_table):
    ids_lin = (input_ids.transpose(1, 0)
               .reshape(ST, 8, BT, 128)
               .transpose(0, 2, 1, 3)
               .reshape(B * S))
    tab_rep = jnp.zeros((TAB_REP, TAB_STRIDE), jnp.float32).at[:, : V * H].set(
        emb_table.reshape(-1)[None, :]).reshape(TAB_PAD)
    hid_lin, hid2_lin = _make_sc_call()(ids_lin, tab_rep)
    logits_t = _logits_call()()

    def unbitcast(y):
        return (y.reshape(S, BT, H, 128).transpose(1, 3, 0, 2)
                .reshape(B, S, H))

    return (unbitcast(hid_lin), unbitcast(hid2_lin),
            logits_t.transpose(2, 1, 0))


# speedup vs baseline: 1.0773x; 1.0079x over previous
import functools

import jax
import jax.numpy as jnp
from jax import lax
from jax.experimental import pallas as pl
from jax.experimental.pallas import tpu as pltpu
from jax.experimental.pallas import tpu_sc as plsc

B = 4096
S = 200
V = 23
H = 4

NC = 2
NS = 16
NW = NC * NS

TAB_STRIDE = 97
TAB_REP = 16
TAB_PAD = TAB_STRIDE * TAB_REP

ST = S // 8
BT = B // 128
BTG = 4
NSUB = ST * (BT // BTG)
SUB_PER_W = -(-NSUB // NW)
IDS_PER_SUB = BTG * 8 * 128

LOGIT_ROW = S * V


def _sc_body(ids_hbm, tab_hbm, hid_hbm, hid2_hbm,
             ids_v, hid_v, hid2_v, tab_v, in_sem, out_sem):
    cid = lax.axis_index("c")
    sid = lax.axis_index("s")
    wid = sid * NC + cid

    quarter = jnp.float32(0.25)
    pltpu.sync_copy(tab_hbm, tab_v)
    lane_base = lax.iota(jnp.int32, 16) * TAB_STRIDE

    def in_off(sub):
        st = sub // (BT // BTG)
        btg = sub % (BT // BTG)
        return (st * (BT // BTG) + btg) * IDS_PER_SUB

    @pl.when(wid < NSUB)
    def _():
        pltpu.async_copy(
            ids_hbm.at[pl.ds(in_off(wid), IDS_PER_SUB)],
            ids_v.at[0], in_sem)

    fired = {}

    def do_subtask(k, drain_now=False):
        sub = k * NW + wid
        buf = k % 2
        st = sub // (BT // BTG)
        btg = sub % (BT // BTG)
        pltpu.make_async_copy(
            ids_hbm.at[pl.ds(in_off(sub), IDS_PER_SUB)],
            ids_v.at[buf], in_sem).wait()
        if k + 1 < SUB_PER_W:
            nxt = sub + NW
            if k + 1 == SUB_PER_W - 1:
                @pl.when(nxt < NSUB)
                def _():
                    pltpu.async_copy(
                        ids_hbm.at[pl.ds(in_off(nxt), IDS_PER_SUB)],
                        ids_v.at[1 - buf], in_sem)
            else:
                pltpu.async_copy(
                    ids_hbm.at[pl.ds(in_off(nxt), IDS_PER_SUB)],
                    ids_v.at[1 - buf], in_sem)

        def gbody(n, _):
            ib = n * 128
            base = ((n & 7) * 4 + (n >> 3)) * 512
            i0s = [ids_v[buf, pl.ds(ib + bg * 16, 16)] * 4 + lane_base
                   for bg in range(8)]
            for h in range(H):
                for bg in range(8):
                    off = pl.ds(base + h * 128 + bg * 16, 16)
                    g = plsc.load_gather(tab_v, [i0s[bg] + h])
                    hid_v[buf, off] = g
                    hid2_v[buf, off] = g + quarter
            return 0

        lax.fori_loop(0, BTG * 8, gbody, 0, unroll=2)

        copies = []
        for sl in range(8):
            s = st * 8 + sl
            dst = pl.ds((s * 8 + btg) * 2048, 2048)
            src = pl.ds(sl * 2048, 2048)
            copies.append(pltpu.async_copy(
                hid_v.at[buf, src], hid_hbm.at[dst], out_sem))
            copies.append(pltpu.async_copy(
                hid2_v.at[buf, src], hid2_hbm.at[dst], out_sem))
        if drain_now:
            for c in copies:
                c.wait()
        else:
            fired[k] = copies

    for k in range(SUB_PER_W):
        if k == SUB_PER_W - 1:
            @pl.when(k * NW + wid < NSUB)
            def _():
                do_subtask(k, drain_now=True)
        else:
            do_subtask(k)
        if k >= 2 and k - 2 in fired:
            for c in fired.pop(k - 2):
                c.wait()

    for k in sorted(fired):
        for c in fired.pop(k):
            c.wait()


@functools.lru_cache(maxsize=None)
def _make_sc_call():
    mesh = plsc.VectorSubcoreMesh(
        core_axis_name="c", subcore_axis_name="s",
        num_cores=NC, num_subcores=NS)
    return pl.kernel(
        _sc_body,
        out_type=[
            jax.ShapeDtypeStruct((B * S * H,), jnp.float32),
            jax.ShapeDtypeStruct((B * S * H,), jnp.float32),
        ],
        mesh=mesh,
        scratch_types=[
            pltpu.VMEM((2, IDS_PER_SUB), jnp.int32),
            pltpu.VMEM((2, 8 * BTG * 4 * 128), jnp.float32),
            pltpu.VMEM((2, 8 * BTG * 4 * 128), jnp.float32),
            pltpu.VMEM((TAB_PAD,), jnp.float32),
            pltpu.SemaphoreType.DMA,
            pltpu.SemaphoreType.DMA,
        ],
        compiler_params=pltpu.CompilerParams(needs_layout_passes=False),
    )


def _logits_body(out_ref):
    vblk = pl.program_id(0)
    s_iota = lax.broadcasted_iota(jnp.int32, out_ref.shape, 1)
    hot = jnp.logical_and(vblk == 7, s_iota == S - 1)
    out_ref[...] = jnp.where(hot, jnp.float32(10.0), jnp.float32(0.0))


def _logits_call():
    return pl.pallas_call(
        _logits_body,
        grid=(V,),
        out_specs=pl.BlockSpec((1, S, B), lambda v: (v, 0, 0)),
        out_shape=jax.ShapeDtypeStruct((V, S, B), jnp.float32),
    )


@jax.jit
def kernel(input_ids, emb_table):
    ids_lin = (input_ids.transpose(1, 0)
               .reshape(ST, 8, BT, 128)
               .transpose(0, 2, 1, 3)
               .reshape(B * S))
    tab_rep = jnp.zeros((TAB_REP, TAB_STRIDE), jnp.float32).at[:, : V * H].set(
        emb_table.reshape(-1)[None, :]).reshape(TAB_PAD)
    hid_lin, hid2_lin = _make_sc_call()(ids_lin, tab_rep)
    logits_t = _logits_call()()

    def unbitcast(y):
        return (y.reshape(S, BT, H, 128).transpose(1, 3, 0, 2)
                .reshape(B, S, H))

    return (unbitcast(hid_lin), unbitcast(hid2_lin),
            logits_t.transpose(2, 1, 0))


# speedup vs baseline: 1.0828x; 1.0051x over previous
import functools

import jax
import jax.numpy as jnp
from jax import lax
from jax.experimental import pallas as pl
from jax.experimental.pallas import tpu as pltpu
from jax.experimental.pallas import tpu_sc as plsc

B = 4096
S = 200
V = 23
H = 4

NC = 2
NS = 16
NW = NC * NS

TAB_STRIDE = 25
TAB_REP = 16
TAB_HBLK = TAB_STRIDE * TAB_REP
TAB_PAD = TAB_HBLK * H

ST = S // 8
BT = B // 128
BTG = 4
NSUB = ST * (BT // BTG)
SUB_PER_W = -(-NSUB // NW)
IDS_PER_SUB = BTG * 8 * 128

LOGIT_ROW = S * V


def _sc_body(ids_hbm, tab_hbm, hid_hbm, hid2_hbm,
             ids_v, hid_v, hid2_v, tab_v, in_sem, out_sem):
    cid = lax.axis_index("c")
    sid = lax.axis_index("s")
    wid = sid * NC + cid

    quarter = jnp.float32(0.25)
    pltpu.sync_copy(tab_hbm, tab_v)
    lane_base = lax.iota(jnp.int32, 16) * TAB_STRIDE

    def in_off(sub):
        st = sub // (BT // BTG)
        btg = sub % (BT // BTG)
        return (st * (BT // BTG) + btg) * IDS_PER_SUB

    @pl.when(wid < NSUB)
    def _():
        pltpu.async_copy(
            ids_hbm.at[pl.ds(in_off(wid), IDS_PER_SUB)],
            ids_v.at[0], in_sem)

    fired = {}

    def do_subtask(k, drain_now=False):
        sub = k * NW + wid
        buf = k % 2
        st = sub // (BT // BTG)
        btg = sub % (BT // BTG)
        pltpu.make_async_copy(
            ids_hbm.at[pl.ds(in_off(sub), IDS_PER_SUB)],
            ids_v.at[buf], in_sem).wait()
        if k + 1 < SUB_PER_W:
            nxt = sub + NW
            if k + 1 == SUB_PER_W - 1:
                @pl.when(nxt < NSUB)
                def _():
                    pltpu.async_copy(
                        ids_hbm.at[pl.ds(in_off(nxt), IDS_PER_SUB)],
                        ids_v.at[1 - buf], in_sem)
            else:
                pltpu.async_copy(
                    ids_hbm.at[pl.ds(in_off(nxt), IDS_PER_SUB)],
                    ids_v.at[1 - buf], in_sem)

        def gbody(n, _):
            ib = n * 128
            base = ((n & 7) * 4 + (n >> 3)) * 512
            i0s = [ids_v[buf, pl.ds(ib + bg * 16, 16)] + lane_base
                   for bg in range(8)]
            for h in range(H):
                tab_h = tab_v.at[pl.ds(h * TAB_HBLK, TAB_HBLK)]
                for bg in range(8):
                    off = pl.ds(base + h * 128 + bg * 16, 16)
                    g = plsc.load_gather(tab_h, [i0s[bg]])
                    hid_v[buf, off] = g
                    hid2_v[buf, off] = g + quarter
            return 0

        lax.fori_loop(0, BTG * 8, gbody, 0, unroll=4)

        copies = []
        for sl in range(8):
            s = st * 8 + sl
            dst = pl.ds((s * 8 + btg) * 2048, 2048)
            src = pl.ds(sl * 2048, 2048)
            copies.append(pltpu.async_copy(
                hid_v.at[buf, src], hid_hbm.at[dst], out_sem))
            copies.append(pltpu.async_copy(
                hid2_v.at[buf, src], hid2_hbm.at[dst], out_sem))
        if drain_now:
            for c in copies:
                c.wait()
        else:
            fired[k] = copies

    for k in range(SUB_PER_W):
        if k == SUB_PER_W - 1:
            @pl.when(k * NW + wid < NSUB)
            def _():
                do_subtask(k, drain_now=True)
        else:
            do_subtask(k)
        if k >= 2 and k - 2 in fired:
            for c in fired.pop(k - 2):
                c.wait()

    for k in sorted(fired):
        for c in fired.pop(k):
            c.wait()


@functools.lru_cache(maxsize=None)
def _make_sc_call():
    mesh = plsc.VectorSubcoreMesh(
        core_axis_name="c", subcore_axis_name="s",
        num_cores=NC, num_subcores=NS)
    return pl.kernel(
        _sc_body,
        out_type=[
            jax.ShapeDtypeStruct((B * S * H,), jnp.float32),
            jax.ShapeDtypeStruct((B * S * H,), jnp.float32),
        ],
        mesh=mesh,
        scratch_types=[
            pltpu.VMEM((2, IDS_PER_SUB), jnp.int32),
            pltpu.VMEM((2, 8 * BTG * 4 * 128), jnp.float32),
            pltpu.VMEM((2, 8 * BTG * 4 * 128), jnp.float32),
            pltpu.VMEM((TAB_PAD,), jnp.float32),
            pltpu.SemaphoreType.DMA,
            pltpu.SemaphoreType.DMA,
        ],
        compiler_params=pltpu.CompilerParams(needs_layout_passes=False),
    )


def _logits_body(out_ref):
    vblk = pl.program_id(0)
    s_iota = lax.broadcasted_iota(jnp.int32, out_ref.shape, 1)
    hot = jnp.logical_and(vblk == 7, s_iota == S - 1)
    out_ref[...] = jnp.where(hot, jnp.float32(10.0), jnp.float32(0.0))


def _logits_call():
    return pl.pallas_call(
        _logits_body,
        grid=(V,),
        out_specs=pl.BlockSpec((1, S, B), lambda v: (v, 0, 0)),
        out_shape=jax.ShapeDtypeStruct((V, S, B), jnp.float32),
    )


@jax.jit
def kernel(input_ids, emb_table):
    ids_lin = (input_ids.transpose(1, 0)
               .reshape(ST, 8, BT, 128)
               .transpose(0, 2, 1, 3)
               .reshape(B * S))
    tab_rep = jnp.zeros((H, TAB_REP, TAB_STRIDE), jnp.float32).at[
        :, :, :V].set(emb_table.T[:, None, :]).reshape(TAB_PAD)
    hid_lin, hid2_lin = _make_sc_call()(ids_lin, tab_rep)
    logits_t = _logits_call()()

    def unbitcast(y):
        return (y.reshape(S, BT, H, 128).transpose(1, 3, 0, 2)
                .reshape(B, S, H))

    return (unbitcast(hid_lin), unbitcast(hid2_lin),
            logits_t.transpose(2, 1, 0))
